# baseline (speedup 1.0000x reference)
import jax
import jax.numpy as jnp
from jax.experimental import pallas as pl

B, P, V = 128, 16, 1024
N = P * P
K = 32
NEG_INF = float("-inf")


def _fused_body(logits_ref, mask_ref, u_sample_ref, u_conf_ref, code_ref,
                code_out_ref, mask_out_ref, tresh_ref):
    x = logits_ref[0]
    u = u_sample_ref[0]
    maskv = mask_ref[0]
    uc = u_conf_ref[0]
    code = code_ref[0]

    m = jnp.max(x, axis=-1, keepdims=True)
    e = jnp.exp(x - m)
    s = jnp.sum(e, axis=-1, keepdims=True)
    p = e / s

    g = -jnp.log(-jnp.log(u + 1e-9) + 1e-9)
    score = jnp.log(p + 1e-12) + g
    pred = jnp.argmax(score, axis=-1).astype(jnp.int32)

    col = jax.lax.broadcasted_iota(jnp.int32, (N, V), 1)
    conf_p = jnp.sum(jnp.where(col == pred[:, None], p, 0.0), axis=-1)

    gc = -jnp.log(-jnp.log(uc + 1e-9) + 1e-9)
    conf = jnp.log(conf_p + 1e-12)[None, :] + 4.5 * gc
    mask_bool = maskv != 0.0
    conf = jnp.where(mask_bool, conf, NEG_INF)

    pos = jax.lax.broadcasted_iota(jnp.int32, (1, N), 1)
    kio = jax.lax.broadcasted_iota(jnp.int32, (1, K), 1)

    def step(t, carry):
        work, mask_f, vals = carry
        mx = jnp.max(work)
        idx = jnp.min(jnp.where(work == mx, pos, jnp.int32(2**30)))
        vals = jnp.where(kio == t, mx, vals)
        hit = pos == idx
        work = jnp.where(hit, NEG_INF, work)
        mask_f = jnp.where(hit, 0.0, mask_f)
        return work, mask_f, vals

    init = (conf, maskv, jnp.zeros((1, K), jnp.float32))
    _, new_mask, vals = jax.lax.fori_loop(0, K, step, init)

    tresh = vals[0, K - 1]
    sel = conf >= tresh
    pred_row = pred[None, :]
    new_code = jnp.where(mask_bool & sel, pred_row, code)

    code_out_ref[0] = new_code
    mask_out_ref[0] = new_mask
    tresh_ref[0] = vals


def kernel(logits, mask, u_sample, u_conf, code, k):
    del k
    code_flat = code.reshape(B, 1, N)
    mask3 = mask.reshape(B, 1, N)
    u_conf3 = u_conf.reshape(B, 1, N)

    grid = (B,)
    out_shapes = (
        jax.ShapeDtypeStruct((B, 1, N), jnp.int32),
        jax.ShapeDtypeStruct((B, 1, N), jnp.float32),
        jax.ShapeDtypeStruct((B, 1, K), jnp.float32),
    )
    new_code, new_mask, tresh_conf = pl.pallas_call(
        _fused_body,
        grid=grid,
        in_specs=[
            pl.BlockSpec((1, N, V), lambda b: (b, 0, 0)),
            pl.BlockSpec((1, 1, N), lambda b: (b, 0, 0)),
            pl.BlockSpec((1, N, V), lambda b: (b, 0, 0)),
            pl.BlockSpec((1, 1, N), lambda b: (b, 0, 0)),
            pl.BlockSpec((1, 1, N), lambda b: (b, 0, 0)),
        ],
        out_specs=(
            pl.BlockSpec((1, 1, N), lambda b: (b, 0, 0)),
            pl.BlockSpec((1, 1, N), lambda b: (b, 0, 0)),
            pl.BlockSpec((1, 1, K), lambda b: (b, 0, 0)),
        ),
        out_shape=out_shapes,
    )(logits, mask3, u_sample, u_conf3, code_flat)

    return (new_code.reshape(B, P, P), new_mask.reshape(B, N),
            tresh_conf.reshape(B, K))


# speedup vs baseline: 7.1690x; 7.1690x over previous
import jax
import jax.numpy as jnp
from jax.experimental import pallas as pl
from jax.experimental.pallas import tpu as pltpu

B, P, V = 128, 16, 1024
N = P * P
K = 32
NEG_INF = float("-inf")


def _dense_body(logits_ref, mask_ref, u_sample_ref, u_conf_ref,
                conf_ref, pred_ref):
    x = logits_ref[0]
    u = u_sample_ref[0]
    maskv = mask_ref[0]
    uc = u_conf_ref[0]

    m = jnp.max(x, axis=-1, keepdims=True)
    e = jnp.exp(x - m)
    s = jnp.sum(e, axis=-1, keepdims=True)
    p = e / s

    g = -jnp.log(-jnp.log(u + 1e-9) + 1e-9)
    score = jnp.log(p + 1e-12) + g
    pred = jnp.argmax(score, axis=-1).astype(jnp.int32)

    col = jax.lax.broadcasted_iota(jnp.int32, (N, V), 1)
    conf_p = jnp.sum(jnp.where(col == pred[:, None], p, 0.0), axis=-1)

    gc = -jnp.log(-jnp.log(uc + 1e-9) + 1e-9)
    conf = jnp.log(conf_p + 1e-12)[None, :] + 4.5 * gc
    conf = jnp.where(maskv != 0.0, conf, NEG_INF)

    conf_ref[0] = conf
    pred_ref[0] = pred[None, :]


def _select_body(conf_ref, mask_ref, pred_ref, code_ref,
                 code_out_ref, mask_out_ref, tresh_ref):
    conf = conf_ref[...]
    maskv = mask_ref[...]
    pred = pred_ref[...]
    code = code_ref[...]

    pos = jax.lax.broadcasted_iota(jnp.int32, (B, N), 1)
    kio = jax.lax.broadcasted_iota(jnp.int32, (B, K), 1)

    def step(t, carry):
        work, mask_f, vals = carry
        mx = jnp.max(work, axis=1, keepdims=True)
        idx = jnp.min(jnp.where(work == mx, pos, jnp.int32(2**30)),
                      axis=1, keepdims=True)
        vals = jnp.where(kio == t, mx, vals)
        hit = pos == idx
        work = jnp.where(hit, NEG_INF, work)
        mask_f = jnp.where(hit, 0.0, mask_f)
        return work, mask_f, vals

    init = (conf, maskv, jnp.zeros((B, K), jnp.float32))
    _, new_mask, vals = jax.lax.fori_loop(0, K, step, init)

    tresh = vals[:, K - 1:K]
    sel = conf >= tresh
    new_code = jnp.where((maskv != 0.0) & sel, pred, code)

    code_out_ref[...] = new_code
    mask_out_ref[...] = new_mask
    tresh_ref[...] = vals


def kernel(logits, mask, u_sample, u_conf, code, k):
    del k
    mask3 = mask.reshape(B, 1, N)
    u_conf3 = u_conf.reshape(B, 1, N)

    conf, pred = pl.pallas_call(
        _dense_body,
        grid=(B,),
        in_specs=[
            pl.BlockSpec((1, N, V), lambda b: (b, 0, 0)),
            pl.BlockSpec((1, 1, N), lambda b: (b, 0, 0)),
            pl.BlockSpec((1, N, V), lambda b: (b, 0, 0)),
            pl.BlockSpec((1, 1, N), lambda b: (b, 0, 0)),
        ],
        out_specs=(
            pl.BlockSpec((1, 1, N), lambda b: (b, 0, 0)),
            pl.BlockSpec((1, 1, N), lambda b: (b, 0, 0)),
        ),
        out_shape=(
            jax.ShapeDtypeStruct((B, 1, N), jnp.float32),
            jax.ShapeDtypeStruct((B, 1, N), jnp.int32),
        ),
        compiler_params=pltpu.CompilerParams(
            dimension_semantics=("parallel",)),
    )(logits, mask3, u_sample, u_conf3)

    new_code, new_mask, tresh_conf = pl.pallas_call(
        _select_body,
        out_shape=(
            jax.ShapeDtypeStruct((B, N), jnp.int32),
            jax.ShapeDtypeStruct((B, N), jnp.float32),
            jax.ShapeDtypeStruct((B, K), jnp.float32),
        ),
    )(conf.reshape(B, N), mask, pred.reshape(B, N), code.reshape(B, N))

    return (new_code.reshape(B, P, P), new_mask, tresh_conf)


# speedup vs baseline: 7.5301x; 1.0504x over previous
import jax
import jax.numpy as jnp
from jax.experimental import pallas as pl
from jax.experimental.pallas import tpu as pltpu

B, P, V = 128, 16, 1024
N = P * P
K = 32
NEG_INF = float("-inf")


def _dense_body(logits_ref, mask_ref, u_sample_ref, u_conf_ref,
                conf_ref, pred_ref):
    x = logits_ref[0]
    u = u_sample_ref[0]
    maskv = mask_ref[0]
    uc = u_conf_ref[0]

    m = jnp.max(x, axis=-1, keepdims=True)
    e = jnp.exp(x - m)
    s = jnp.sum(e, axis=-1, keepdims=True)

    den = -jnp.log(u + 1e-9) + 1e-9
    r = (e + 1e-12 * s) / den
    pred = jnp.argmax(r, axis=-1).astype(jnp.int32)

    col = jax.lax.broadcasted_iota(jnp.int32, (N, V), 1)
    e_pred = jnp.sum(jnp.where(col == pred[:, None], e, 0.0), axis=-1)
    conf_p = e_pred / s[:, 0]

    gc = -jnp.log(-jnp.log(uc + 1e-9) + 1e-9)
    conf = jnp.log(conf_p + 1e-12)[None, :] + 4.5 * gc
    conf = jnp.where(maskv != 0.0, conf, NEG_INF)

    conf_ref[0] = conf
    pred_ref[0] = pred[None, :]


def _select_body(conf_ref, mask_ref, pred_ref, code_ref,
                 code_out_ref, mask_out_ref, tresh_ref):
    conf = conf_ref[...]
    maskv = mask_ref[...]
    pred = pred_ref[...]
    code = code_ref[...]

    pos = jax.lax.broadcasted_iota(jnp.int32, (B, N), 1)
    kio = jax.lax.broadcasted_iota(jnp.int32, (B, K), 1)

    def step(t, carry):
        work, mask_f, vals = carry
        mx = jnp.max(work, axis=1, keepdims=True)
        idx = jnp.min(jnp.where(work == mx, pos, jnp.int32(2**30)),
                      axis=1, keepdims=True)
        vals = jnp.where(kio == t, mx, vals)
        hit = pos == idx
        work = jnp.where(hit, NEG_INF, work)
        mask_f = jnp.where(hit, 0.0, mask_f)
        return work, mask_f, vals

    init = (conf, maskv, jnp.zeros((B, K), jnp.float32))
    _, new_mask, vals = jax.lax.fori_loop(0, K, step, init)

    tresh = vals[:, K - 1:K]
    sel = conf >= tresh
    new_code = jnp.where((maskv != 0.0) & sel, pred, code)

    code_out_ref[...] = new_code
    mask_out_ref[...] = new_mask
    tresh_ref[...] = vals


def kernel(logits, mask, u_sample, u_conf, code, k):
    del k
    mask3 = mask.reshape(B, 1, N)
    u_conf3 = u_conf.reshape(B, 1, N)

    conf, pred = pl.pallas_call(
        _dense_body,
        grid=(B,),
        in_specs=[
            pl.BlockSpec((1, N, V), lambda b: (b, 0, 0)),
            pl.BlockSpec((1, 1, N), lambda b: (b, 0, 0)),
            pl.BlockSpec((1, N, V), lambda b: (b, 0, 0)),
            pl.BlockSpec((1, 1, N), lambda b: (b, 0, 0)),
        ],
        out_specs=(
            pl.BlockSpec((1, 1, N), lambda b: (b, 0, 0)),
            pl.BlockSpec((1, 1, N), lambda b: (b, 0, 0)),
        ),
        out_shape=(
            jax.ShapeDtypeStruct((B, 1, N), jnp.float32),
            jax.ShapeDtypeStruct((B, 1, N), jnp.int32),
        ),
        compiler_params=pltpu.CompilerParams(
            dimension_semantics=("parallel",)),
    )(logits, mask3, u_sample, u_conf3)

    new_code, new_mask, tresh_conf = pl.pallas_call(
        _select_body,
        out_shape=(
            jax.ShapeDtypeStruct((B, N), jnp.int32),
            jax.ShapeDtypeStruct((B, N), jnp.float32),
            jax.ShapeDtypeStruct((B, K), jnp.float32),
        ),
    )(conf.reshape(B, N), mask, pred.reshape(B, N), code.reshape(B, N))

    return (new_code.reshape(B, P, P), new_mask, tresh_conf)


# speedup vs baseline: 10.0556x; 1.3354x over previous
import jax
import jax.numpy as jnp
from jax.experimental import pallas as pl
from jax.experimental.pallas import tpu as pltpu

B, P, V = 128, 16, 1024
N = P * P
K = 32
NEG_INF = float("-inf")


def _dense_body(logits_ref, mask_ref, u_sample_ref, u_conf_ref,
                conf_ref, pred_ref):
    x = logits_ref[...]
    u = u_sample_ref[...]
    maskv = mask_ref[0]
    uc = u_conf_ref[0]

    m = jnp.max(x, axis=-1, keepdims=True)
    e = jnp.exp(x - m)
    s = jnp.sum(e, axis=-1, keepdims=True)

    den = -jnp.log(u + 1e-9) + 1e-9
    r = (e + 1e-12 * s) / den
    pred = jnp.argmax(r, axis=-1).astype(jnp.int32)

    col = jax.lax.broadcasted_iota(jnp.int32, x.shape, 1)
    e_pred = jnp.sum(jnp.where(col == pred[:, None], e, 0.0), axis=-1)
    conf_p = e_pred / s[:, 0]

    gc = -jnp.log(-jnp.log(uc + 1e-9) + 1e-9)
    conf = jnp.log(conf_p + 1e-12)[None, :] + 4.5 * gc
    conf = jnp.where(maskv != 0.0, conf, NEG_INF)

    conf_ref[0] = conf
    pred_ref[0] = pred[None, :]


def _select_body(conf_ref, mask_ref, pred_ref, code_ref,
                 code_out_ref, mask_out_ref, tresh_ref):
    conf = conf_ref[...]
    maskv = mask_ref[...]
    pred = pred_ref[...]
    code = code_ref[...]

    pos = jax.lax.broadcasted_iota(jnp.int32, (B, N), 1)
    kio = jax.lax.broadcasted_iota(jnp.int32, (B, K), 1)

    def step(t, carry):
        work, mask_f, vals = carry
        mx = jnp.max(work, axis=1, keepdims=True)
        idx = jnp.min(jnp.where(work == mx, pos, jnp.int32(2**30)),
                      axis=1, keepdims=True)
        vals = jnp.where(kio == t, mx, vals)
        hit = pos == idx
        work = jnp.where(hit, NEG_INF, work)
        mask_f = jnp.where(hit, 0.0, mask_f)
        return work, mask_f, vals

    init = (conf, maskv, jnp.zeros((B, K), jnp.float32))
    _, new_mask, vals = jax.lax.fori_loop(0, K, step, init)

    tresh = vals[:, K - 1:K]
    sel = conf >= tresh
    new_code = jnp.where((maskv != 0.0) & sel, pred, code)

    code_out_ref[...] = new_code
    mask_out_ref[...] = new_mask
    tresh_ref[...] = vals


G = 32
RB = (B * N) // G


def kernel(logits, mask, u_sample, u_conf, code, k):
    del k
    logits2 = logits.reshape(B * N, V)
    u_sample2 = u_sample.reshape(B * N, V)
    mask3 = mask.reshape(G, 1, RB)
    u_conf3 = u_conf.reshape(G, 1, RB)

    conf, pred = pl.pallas_call(
        _dense_body,
        grid=(G,),
        in_specs=[
            pl.BlockSpec((RB, V), lambda b: (b, 0)),
            pl.BlockSpec((1, 1, RB), lambda b: (b, 0, 0)),
            pl.BlockSpec((RB, V), lambda b: (b, 0)),
            pl.BlockSpec((1, 1, RB), lambda b: (b, 0, 0)),
        ],
        out_specs=(
            pl.BlockSpec((1, 1, RB), lambda b: (b, 0, 0)),
            pl.BlockSpec((1, 1, RB), lambda b: (b, 0, 0)),
        ),
        out_shape=(
            jax.ShapeDtypeStruct((G, 1, RB), jnp.float32),
            jax.ShapeDtypeStruct((G, 1, RB), jnp.int32),
        ),
        compiler_params=pltpu.CompilerParams(
            dimension_semantics=("parallel",)),
    )(logits2, mask3, u_sample2, u_conf3)

    new_code, new_mask, tresh_conf = pl.pallas_call(
        _select_body,
        out_shape=(
            jax.ShapeDtypeStruct((B, N), jnp.int32),
            jax.ShapeDtypeStruct((B, N), jnp.float32),
            jax.ShapeDtypeStruct((B, K), jnp.float32),
        ),
    )(conf.reshape(B, N), mask, pred.reshape(B, N), code.reshape(B, N))

    return (new_code.reshape(B, P, P), new_mask, tresh_conf)


# speedup vs baseline: 10.0940x; 1.0038x over previous
import jax
import jax.numpy as jnp
from jax.experimental import pallas as pl
from jax.experimental.pallas import tpu as pltpu

B, P, V = 128, 16, 1024
N = P * P
K = 32
NEG_INF = float("-inf")


def _dense_body(logits_ref, mask_ref, u_sample_ref, u_conf_ref,
                conf_ref, pred_ref):
    x = logits_ref[...]
    u = u_sample_ref[...]
    maskv = mask_ref[0]
    uc = u_conf_ref[0]

    m = jnp.max(x, axis=-1, keepdims=True)
    e = jnp.exp(x - m)
    s = jnp.sum(e, axis=-1, keepdims=True)

    den = -jnp.log(u + 1e-9) + 1e-9
    r = (e + 1e-12 * s) / den
    pred = jnp.argmax(r, axis=-1).astype(jnp.int32)

    col = jax.lax.broadcasted_iota(jnp.int32, x.shape, 1)
    e_pred = jnp.sum(jnp.where(col == pred[:, None], e, 0.0), axis=-1)
    conf_p = e_pred / s[:, 0]

    gc = -jnp.log(-jnp.log(uc + 1e-9) + 1e-9)
    conf = jnp.log(conf_p + 1e-12)[None, :] + 4.5 * gc
    conf = jnp.where(maskv != 0.0, conf, NEG_INF)

    conf_ref[0] = conf
    pred_ref[0] = pred[None, :]


def _select_body(conf_ref, mask_ref, pred_ref, code_ref,
                 code_out_ref, mask_out_ref, tresh_ref):
    conf = conf_ref[...]
    maskv = mask_ref[...]
    pred = pred_ref[...]
    code = code_ref[...]

    pos = jax.lax.broadcasted_iota(jnp.int32, (B, N), 1)
    kio = jax.lax.broadcasted_iota(jnp.int32, (B, K), 1)

    def step(t, carry):
        work, mask_f, vals = carry
        mx = jnp.max(work, axis=1, keepdims=True)
        idx = jnp.min(jnp.where(work == mx, pos, jnp.int32(2**30)),
                      axis=1, keepdims=True)
        vals = jnp.where(kio == t, mx, vals)
        hit = pos == idx
        work = jnp.where(hit, NEG_INF, work)
        mask_f = jnp.where(hit, 0.0, mask_f)
        return work, mask_f, vals

    init = (conf, maskv, jnp.zeros((B, K), jnp.float32))
    _, new_mask, vals = jax.lax.fori_loop(0, K, step, init)

    tresh = vals[:, K - 1:K]
    sel = conf >= tresh
    new_code = jnp.where((maskv != 0.0) & sel, pred, code)

    code_out_ref[...] = new_code
    mask_out_ref[...] = new_mask
    tresh_ref[...] = vals


G = 32
RB = (B * N) // G


def kernel(logits, mask, u_sample, u_conf, code, k):
    del k
    logits2 = logits.reshape(B * N, V)
    u_sample2 = u_sample.reshape(B * N, V)
    mask3 = mask.reshape(G, 1, RB)
    u_conf3 = u_conf.reshape(G, 1, RB)

    conf, pred = pl.pallas_call(
        _dense_body,
        grid=(G,),
        in_specs=[
            pl.BlockSpec((RB, V), lambda b: (b, 0)),
            pl.BlockSpec((1, 1, RB), lambda b: (b, 0, 0)),
            pl.BlockSpec((RB, V), lambda b: (b, 0)),
            pl.BlockSpec((1, 1, RB), lambda b: (b, 0, 0)),
        ],
        out_specs=(
            pl.BlockSpec((1, 1, RB), lambda b: (b, 0, 0)),
            pl.BlockSpec((1, 1, RB), lambda b: (b, 0, 0)),
        ),
        out_shape=(
            jax.ShapeDtypeStruct((G, 1, RB), jnp.float32),
            jax.ShapeDtypeStruct((G, 1, RB), jnp.int32),
        ),
        compiler_params=pltpu.CompilerParams(
            dimension_semantics=("arbitrary",)),
    )(logits2, mask3, u_sample2, u_conf3)

    new_code, new_mask, tresh_conf = pl.pallas_call(
        _select_body,
        out_shape=(
            jax.ShapeDtypeStruct((B, N), jnp.int32),
            jax.ShapeDtypeStruct((B, N), jnp.float32),
            jax.ShapeDtypeStruct((B, K), jnp.float32),
        ),
    )(conf.reshape(B, N), mask, pred.reshape(B, N), code.reshape(B, N))

    return (new_code.reshape(B, P, P), new_mask, tresh_conf)
